# baseline (speedup 1.0000x reference)
import jax
import jax.numpy as jnp
from jax import lax
from jax.experimental import pallas as pl
from jax.experimental.pallas import tpu as pltpu
from jax.experimental.pallas import tpu_sc as plsc

N = 10000
N_PAD = 10240
D = 128
E = 320000
NEG = 0.2

NC, NS = 2, 16
NW = NC * NS
EPW = E // NW
C = 40
NCH = EPW // C
RPT = N_PAD // NS
LJ = D // 16
LANES = 16
GROUPS = ((0, 0), (16, 0), (24, 8))


def _lane_gather(x, idx):
  return lax.gather(
      x, idx[:, None],
      lax.GatherDimensionNumbers(offset_dims=(), collapsed_slice_dims=(0,),
                                 start_index_map=(0,)),
      slice_sizes=(1,),
      mode=lax.GatherScatterMode.PROMISE_IN_BOUNDS)


def _sc_edge_body(xl, xr, src, dst, att,
                  acc_out, den_out,
                  att_v,
                  si0, si1, di0, di1,
                  ra0, ra1, rb0, rb1, den_t,
                  smi0, smi1, smg0, smg1, acc_sh):
  sis, dis = [si0, si1], [di0, di1]
  ras, rbs = [ra0, ra1], [rb0, rb1]
  smis, smgs = [smi0, smi1], [smg0, smg1]
  cid = lax.axis_index("c")
  sid = lax.axis_index("s")
  wid = cid * NS + sid
  zero16 = jnp.zeros((LANES,), jnp.float32)
  zero16i = jnp.zeros((LANES,), jnp.int32)
  iota16 = jnp.arange(LANES, dtype=jnp.int32)

  def _zrow(i, carry):
    for j in range(LJ):
      ra0[i, pl.ds(16 * j, 16)] = zero16
    return carry
  lax.fori_loop(0, C, _zrow, 0)
  row0 = sid * RPT
  for k in range(RPT // C):
    pltpu.sync_copy(ra0, acc_sh.at[pl.ds(row0 + k * C, C)])

  def _zden(i, carry):
    den_t[0, pl.ds(16 * i, 16)] = zero16
    return carry
  lax.fori_loop(0, N_PAD // 16, _zden, 0)

  pltpu.sync_copy(att, att_v)
  attv = [att_v[pl.ds(16 * j, 16)] for j in range(LJ)]
  plsc.subcore_barrier()

  ebase = wid * EPW

  def fire_idx(k, p):
    base = ebase + k * C
    pltpu.async_copy(src.at[pl.ds(base, C)], sis[p], smis[p])
    pltpu.async_copy(dst.at[pl.ds(base, C)], dis[p], smis[p])

  def wait_idx(p):
    pltpu.make_async_copy(src.at[pl.ds(0, C)], sis[p], smis[p]).wait()
    pltpu.make_async_copy(dst.at[pl.ds(0, C)], dis[p], smis[p]).wait()

  def fire_gather(pi, pr):
    pltpu.async_copy(xl.at[sis[pi]], ras[pr], smgs[pr])
    pltpu.async_copy(xr.at[dis[pi]], rbs[pr], smgs[pr])

  def wait_gather(pi, pr):
    pltpu.make_async_copy(xl.at[sis[pi]], ras[pr], smgs[pr]).wait()
    pltpu.make_async_copy(xr.at[dis[pi]], rbs[pr], smgs[pr]).wait()

  def compute_chunk(pi, pr):
    ra, rb, dv = ras[pr], rbs[pr], dis[pi]

    @plsc.parallel_loop(0, C, step=1, unroll=8)
    def _edge(e):
      a = [ra[e, pl.ds(16 * j, 16)] for j in range(LJ)]
      acc = zero16
      for j in range(LJ):
        t = a[j] + rb[e, pl.ds(16 * j, 16)]
        acc = acc + attv[j] * jnp.maximum(t, NEG * t)
      sv = jnp.exp(jnp.broadcast_to(jnp.sum(acc), (LANES,)))
      for j in range(LJ):
        ra[e, pl.ds(16 * j, 16)] = a[j] * sv
      rb[e, pl.ds(0, 16)] = sv

    for start, vfrom in GROUPS:
      did = dv[pl.ds(start, LANES)]
      svals = plsc.load_gather(rb, [iota16 + start, zero16i])
      if vfrom:
        svals = jnp.where(iota16 >= vfrom, svals, 0.0)
      ks, vs = plsc.sort_key_val(did, svals)
      for d in (1, 2, 4, 8):
        pidx = jnp.maximum(iota16 - d, 0)
        pk = _lane_gather(ks, pidx)
        pv = _lane_gather(vs, pidx)
        take = jnp.logical_and(iota16 >= d, pk == ks)
        vs = vs + jnp.where(take, pv, 0.0)
      nk = _lane_gather(ks, jnp.minimum(iota16 + 1, LANES - 1))
      is_last = jnp.logical_or(iota16 == LANES - 1, nk != ks)
      plsc.addupdate_scatter(den_t, [zero16i, ks], vs, mask=is_last)

    pltpu.sync_copy(ra, acc_sh.at[dv], add=True)

  fire_idx(0, 0)
  fire_idx(1, 1)
  wait_idx(0)
  fire_gather(0, 0)
  wait_idx(1)
  fire_gather(1, 1)
  wait_gather(0, 0)
  compute_chunk(0, 0)
  fire_idx(2, 0)

  def _body(j, carry):
    for p in range(2):
      k = 1 + 2 * j + p
      cur, nxt = (1 + p) % 2, p
      wait_idx(nxt)
      fire_gather(nxt, nxt)
      wait_gather(cur, cur)
      compute_chunk(cur, cur)

      @pl.when(k + 2 < NCH)
      def _():
        fire_idx(k + 2, cur)
    return carry
  lax.fori_loop(0, (NCH - 2) // 2, _body, 0)

  wait_gather((NCH - 1) % 2, (NCH - 1) % 2)
  compute_chunk((NCH - 1) % 2, (NCH - 1) % 2)

  pltpu.sync_copy(den_t, den_out.at[wid])
  plsc.subcore_barrier()
  for k in range(RPT // C):
    r = row0 + k * C
    pltpu.sync_copy(acc_sh.at[pl.ds(r, C)], ra0)
    pltpu.sync_copy(ra0, acc_out.at[cid, pl.ds(r, C)])


_sc_edge = pl.kernel(
    _sc_edge_body,
    compiler_params=pltpu.CompilerParams(needs_layout_passes=False),
    out_type=(jax.ShapeDtypeStruct((NC, N_PAD, D), jnp.float32),
              jax.ShapeDtypeStruct((NW, 1, N_PAD), jnp.float32)),
    mesh=plsc.VectorSubcoreMesh(core_axis_name="c", subcore_axis_name="s"),
    scratch_types=(
        [pltpu.VMEM((D,), jnp.float32)]
        + [pltpu.VMEM((C,), jnp.int32)] * 4
        + [pltpu.VMEM((C, D), jnp.float32)] * 4
        + [pltpu.VMEM((1, N_PAD), jnp.float32)]
        + [pltpu.SemaphoreType.DMA] * 4
        + [pltpu.VMEM_SHARED((N_PAD, D), jnp.float32)]
    ),
)


BR = 1024


def _proj_body(x_ref, wl_ref, wr_ref, xl_ref, xr_ref):
  x = x_ref[...]
  xl_ref[...] = jnp.dot(x, wl_ref[...], preferred_element_type=jnp.float32)
  xr_ref[...] = jnp.dot(x, wr_ref[...], preferred_element_type=jnp.float32)


def _proj(x, wl, wr):
  return pl.pallas_call(
      _proj_body,
      grid=(N_PAD // BR,),
      in_specs=[pl.BlockSpec((BR, D), lambda i: (i, 0)),
                pl.BlockSpec((D, D), lambda i: (0, 0)),
                pl.BlockSpec((D, D), lambda i: (0, 0))],
      out_specs=[pl.BlockSpec((BR, D), lambda i: (i, 0))] * 2,
      out_shape=(jax.ShapeDtypeStruct((N_PAD, D), jnp.float32),) * 2,
  )(x, wl, wr)


def _x1_of(acc0, acc1, den32, xl, xr, att, b):
  t = xl + xr
  lr = jnp.maximum(t, NEG * t)
  s_self = jnp.exp(jnp.dot(lr, att, preferred_element_type=jnp.float32))
  den_n = lax.dot_general(den32, jnp.ones((NW, 1), jnp.float32),
                          (((0,), (0,)), ((), ())),
                          preferred_element_type=jnp.float32)
  dtot = den_n + s_self + 1e-16
  num = acc0 + acc1 + s_self * xl
  return num / dtot + b


def _acc_specs():
  return [pl.BlockSpec((1, BR, D), lambda i: (0, i, 0)),
          pl.BlockSpec((1, BR, D), lambda i: (1, i, 0)),
          pl.BlockSpec((NW, 1, BR), lambda i: (0, 0, i))]


def _mid_body(acc0_ref, acc1_ref, den_ref, xl_ref, xr_ref,
              att_ref, b_ref, wl2_ref, wr2_ref, xl2_ref, xr2_ref):
  x1 = _x1_of(acc0_ref[0], acc1_ref[0], den_ref[:, 0, :],
              xl_ref[...], xr_ref[...], att_ref[...], b_ref[...])
  xl2_ref[...] = jnp.dot(x1, wl2_ref[...], preferred_element_type=jnp.float32)
  xr2_ref[...] = jnp.dot(x1, wr2_ref[...], preferred_element_type=jnp.float32)


def _mid(acc, den, xl, xr, att, b, wl2, wr2):
  full = lambda r, c: pl.BlockSpec((r, c), lambda i: (0, 0))
  return pl.pallas_call(
      _mid_body,
      grid=(N_PAD // BR,),
      in_specs=_acc_specs() + [
                pl.BlockSpec((BR, D), lambda i: (i, 0)),
                pl.BlockSpec((BR, D), lambda i: (i, 0)),
                full(D, 1), full(1, D), full(D, D), full(D, D)],
      out_specs=[pl.BlockSpec((BR, D), lambda i: (i, 0))] * 2,
      out_shape=(jax.ShapeDtypeStruct((N_PAD, D), jnp.float32),) * 2,
  )(acc, acc, den, xl, xr, att, b, wl2, wr2)


def _fin_body(acc0_ref, acc1_ref, den_ref, xl_ref, xr_ref,
              att_ref, b_ref, x_ref, out_ref):
  x2 = _x1_of(acc0_ref[0], acc1_ref[0], den_ref[:, 0, :],
              xl_ref[...], xr_ref[...], att_ref[...], b_ref[...])
  out_ref[...] = (x2 + x_ref[...]) * 0.5


def _fin(acc, den, xl, xr, att, b, x):
  full = lambda r, c: pl.BlockSpec((r, c), lambda i: (0, 0))
  return pl.pallas_call(
      _fin_body,
      grid=(N_PAD // BR,),
      in_specs=_acc_specs() + [
                pl.BlockSpec((BR, D), lambda i: (i, 0)),
                pl.BlockSpec((BR, D), lambda i: (i, 0)),
                full(D, 1), full(1, D),
                pl.BlockSpec((BR, D), lambda i: (i, 0))],
      out_specs=pl.BlockSpec((BR, D), lambda i: (i, 0)),
      out_shape=jax.ShapeDtypeStruct((N_PAD, D), jnp.float32),
  )(acc, acc, den, xl, xr, att, b, x)


def kernel(input, edge_index, Wl1, Wr1, att1, b1, Wl2, Wr2, att2, b2):
  src = edge_index[0].astype(jnp.int32)
  dst = edge_index[1].astype(jnp.int32)
  x = jnp.pad(input, ((0, N_PAD - N), (0, 0)))

  xl1, xr1 = _proj(x, Wl1, Wr1)
  acc_l1, den_l1 = _sc_edge(xl1, xr1, src, dst, att1)
  xl2, xr2 = _mid(acc_l1, den_l1, xl1, xr1,
                  att1.reshape(D, 1), b1.reshape(1, D), Wl2, Wr2)
  acc_l2, den_l2 = _sc_edge(xl2, xr2, src, dst, att2)
  out = _fin(acc_l2, den_l2, xl2, xr2,
             att2.reshape(D, 1), b2.reshape(1, D), x)
  return out[:N]


# speedup vs baseline: 1.1413x; 1.1413x over previous
import jax
import jax.numpy as jnp
from jax import lax
from jax.experimental import pallas as pl
from jax.experimental.pallas import tpu as pltpu
from jax.experimental.pallas import tpu_sc as plsc

N = 10000
N_PAD = 10240
D = 128
E = 320000
NEG = 0.2

NC, NS = 2, 16
NW = NC * NS
EPW = E // NW
C = 40
NCH = EPW // C
RPT = N_PAD // NS
LJ = D // 16
LANES = 16
GROUPS = ((0, 0), (16, 0), (24, 8))


def _lane_gather(x, idx):
  return lax.gather(
      x, idx[:, None],
      lax.GatherDimensionNumbers(offset_dims=(), collapsed_slice_dims=(0,),
                                 start_index_map=(0,)),
      slice_sizes=(1,),
      mode=lax.GatherScatterMode.PROMISE_IN_BOUNDS)


def _sc_edge_body(xl, xr, src, dst, att,
                  acc_out, den_out,
                  att_v,
                  si0, si1, di0, di1,
                  ra0, ra1, rb0, rb1, den_t,
                  smi0, smi1, smg0, smg1, smsc, acc_sh):
  sis, dis = [si0, si1], [di0, di1]
  ras, rbs = [ra0, ra1], [rb0, rb1]
  smis, smgs = [smi0, smi1], [smg0, smg1]
  cid = lax.axis_index("c")
  sid = lax.axis_index("s")
  wid = cid * NS + sid
  zero16 = jnp.zeros((LANES,), jnp.float32)
  zero16i = jnp.zeros((LANES,), jnp.int32)
  iota16 = jnp.arange(LANES, dtype=jnp.int32)

  def _zrow(i, carry):
    for j in range(LJ):
      ra0[i, pl.ds(16 * j, 16)] = zero16
    return carry
  lax.fori_loop(0, C, _zrow, 0)
  row0 = sid * RPT
  for k in range(RPT // C):
    pltpu.sync_copy(ra0, acc_sh.at[pl.ds(row0 + k * C, C)])

  def _zden(i, carry):
    den_t[0, pl.ds(16 * i, 16)] = zero16
    return carry
  lax.fori_loop(0, N_PAD // 16, _zden, 0)

  pltpu.sync_copy(att, att_v)
  attv = [att_v[pl.ds(16 * j, 16)] for j in range(LJ)]
  plsc.subcore_barrier()

  ebase = wid * EPW

  def fire_idx(k, p):
    base = ebase + k * C
    pltpu.async_copy(src.at[pl.ds(base, C)], sis[p], smis[p])
    pltpu.async_copy(dst.at[pl.ds(base, C)], dis[p], smis[p])

  def wait_idx(p):
    pltpu.make_async_copy(src.at[pl.ds(0, C)], sis[p], smis[p]).wait()
    pltpu.make_async_copy(dst.at[pl.ds(0, C)], dis[p], smis[p]).wait()

  def fire_gather(pi, pr):
    pltpu.async_copy(xl.at[sis[pi]], ras[pr], smgs[pr])
    pltpu.async_copy(xr.at[dis[pi]], rbs[pr], smgs[pr])

  def wait_gather(pi, pr):
    pltpu.make_async_copy(xl.at[sis[pi]], ras[pr], smgs[pr]).wait()
    pltpu.make_async_copy(xr.at[dis[pi]], rbs[pr], smgs[pr]).wait()

  def compute_chunk(pi, pr):
    ra, rb, dv = ras[pr], rbs[pr], dis[pi]

    @plsc.parallel_loop(0, C, step=1, unroll=4)
    def _edge(e):
      a = [ra[e, pl.ds(16 * j, 16)] for j in range(LJ)]
      acc = zero16
      for j in range(LJ):
        t = a[j] + rb[e, pl.ds(16 * j, 16)]
        acc = acc + attv[j] * jnp.maximum(t, NEG * t)
      sv = jnp.exp(jnp.broadcast_to(jnp.sum(acc), (LANES,)))
      for j in range(LJ):
        ra[e, pl.ds(16 * j, 16)] = a[j] * sv
      rb[e, pl.ds(0, 16)] = sv

    sc = pltpu.async_copy(ra, acc_sh.at[dv], smsc, add=True)

    for start, vfrom in GROUPS:
      did = dv[pl.ds(start, LANES)]
      svals = plsc.load_gather(rb, [iota16 + start, zero16i])
      if vfrom:
        svals = jnp.where(iota16 >= vfrom, svals, 0.0)
      ks, vs = plsc.sort_key_val(did, svals)
      for d in (1, 2, 4, 8):
        pidx = jnp.maximum(iota16 - d, 0)
        pk = _lane_gather(ks, pidx)
        pv = _lane_gather(vs, pidx)
        take = jnp.logical_and(iota16 >= d, pk == ks)
        vs = vs + jnp.where(take, pv, 0.0)
      nk = _lane_gather(ks, jnp.minimum(iota16 + 1, LANES - 1))
      is_last = jnp.logical_or(iota16 == LANES - 1, nk != ks)
      plsc.addupdate_scatter(den_t, [zero16i, ks], vs, mask=is_last)

    sc.wait()

  fire_idx(0, 0)
  fire_idx(1, 1)
  wait_idx(0)
  fire_gather(0, 0)
  wait_idx(1)
  fire_gather(1, 1)
  wait_gather(0, 0)
  compute_chunk(0, 0)
  fire_idx(2, 0)

  def _body(j, carry):
    for p in range(2):
      k = 1 + 2 * j + p
      cur, nxt = (1 + p) % 2, p
      wait_idx(nxt)
      fire_gather(nxt, nxt)
      wait_gather(cur, cur)
      compute_chunk(cur, cur)

      @pl.when(k + 2 < NCH)
      def _():
        fire_idx(k + 2, cur)
    return carry
  lax.fori_loop(0, (NCH - 2) // 2, _body, 0)

  wait_gather((NCH - 1) % 2, (NCH - 1) % 2)
  compute_chunk((NCH - 1) % 2, (NCH - 1) % 2)

  pltpu.sync_copy(den_t, den_out.at[wid])
  plsc.subcore_barrier()
  for k in range(RPT // C):
    r = row0 + k * C
    pltpu.sync_copy(acc_sh.at[pl.ds(r, C)], ra0)
    pltpu.sync_copy(ra0, acc_out.at[cid, pl.ds(r, C)])


_sc_edge = pl.kernel(
    _sc_edge_body,
    compiler_params=pltpu.CompilerParams(needs_layout_passes=False),
    out_type=(jax.ShapeDtypeStruct((NC, N_PAD, D), jnp.float32),
              jax.ShapeDtypeStruct((NW, 1, N_PAD), jnp.float32)),
    mesh=plsc.VectorSubcoreMesh(core_axis_name="c", subcore_axis_name="s"),
    scratch_types=(
        [pltpu.VMEM((D,), jnp.float32)]
        + [pltpu.VMEM((C,), jnp.int32)] * 4
        + [pltpu.VMEM((C, D), jnp.float32)] * 4
        + [pltpu.VMEM((1, N_PAD), jnp.float32)]
        + [pltpu.SemaphoreType.DMA] * 5
        + [pltpu.VMEM_SHARED((N_PAD, D), jnp.float32)]
    ),
)


BR = 1024


def _proj_body(x_ref, wl_ref, wr_ref, xl_ref, xr_ref):
  x = x_ref[...]
  xl_ref[...] = jnp.dot(x, wl_ref[...], preferred_element_type=jnp.float32)
  xr_ref[...] = jnp.dot(x, wr_ref[...], preferred_element_type=jnp.float32)


def _proj(x, wl, wr):
  return pl.pallas_call(
      _proj_body,
      grid=(N_PAD // BR,),
      in_specs=[pl.BlockSpec((BR, D), lambda i: (i, 0)),
                pl.BlockSpec((D, D), lambda i: (0, 0)),
                pl.BlockSpec((D, D), lambda i: (0, 0))],
      out_specs=[pl.BlockSpec((BR, D), lambda i: (i, 0))] * 2,
      out_shape=(jax.ShapeDtypeStruct((N_PAD, D), jnp.float32),) * 2,
  )(x, wl, wr)


def _x1_of(acc0, acc1, den32, xl, xr, att, b):
  t = xl + xr
  lr = jnp.maximum(t, NEG * t)
  s_self = jnp.exp(jnp.dot(lr, att, preferred_element_type=jnp.float32))
  den_n = lax.dot_general(den32, jnp.ones((NW, 1), jnp.float32),
                          (((0,), (0,)), ((), ())),
                          preferred_element_type=jnp.float32)
  dtot = den_n + s_self + 1e-16
  num = acc0 + acc1 + s_self * xl
  return num / dtot + b


def _acc_specs():
  return [pl.BlockSpec((1, BR, D), lambda i: (0, i, 0)),
          pl.BlockSpec((1, BR, D), lambda i: (1, i, 0)),
          pl.BlockSpec((NW, 1, BR), lambda i: (0, 0, i))]


def _mid_body(acc0_ref, acc1_ref, den_ref, xl_ref, xr_ref,
              att_ref, b_ref, wl2_ref, wr2_ref, xl2_ref, xr2_ref):
  x1 = _x1_of(acc0_ref[0], acc1_ref[0], den_ref[:, 0, :],
              xl_ref[...], xr_ref[...], att_ref[...], b_ref[...])
  xl2_ref[...] = jnp.dot(x1, wl2_ref[...], preferred_element_type=jnp.float32)
  xr2_ref[...] = jnp.dot(x1, wr2_ref[...], preferred_element_type=jnp.float32)


def _mid(acc, den, xl, xr, att, b, wl2, wr2):
  full = lambda r, c: pl.BlockSpec((r, c), lambda i: (0, 0))
  return pl.pallas_call(
      _mid_body,
      grid=(N_PAD // BR,),
      in_specs=_acc_specs() + [
                pl.BlockSpec((BR, D), lambda i: (i, 0)),
                pl.BlockSpec((BR, D), lambda i: (i, 0)),
                full(D, 1), full(1, D), full(D, D), full(D, D)],
      out_specs=[pl.BlockSpec((BR, D), lambda i: (i, 0))] * 2,
      out_shape=(jax.ShapeDtypeStruct((N_PAD, D), jnp.float32),) * 2,
  )(acc, acc, den, xl, xr, att, b, wl2, wr2)


def _fin_body(acc0_ref, acc1_ref, den_ref, xl_ref, xr_ref,
              att_ref, b_ref, x_ref, out_ref):
  x2 = _x1_of(acc0_ref[0], acc1_ref[0], den_ref[:, 0, :],
              xl_ref[...], xr_ref[...], att_ref[...], b_ref[...])
  out_ref[...] = (x2 + x_ref[...]) * 0.5


def _fin(acc, den, xl, xr, att, b, x):
  full = lambda r, c: pl.BlockSpec((r, c), lambda i: (0, 0))
  return pl.pallas_call(
      _fin_body,
      grid=(N_PAD // BR,),
      in_specs=_acc_specs() + [
                pl.BlockSpec((BR, D), lambda i: (i, 0)),
                pl.BlockSpec((BR, D), lambda i: (i, 0)),
                full(D, 1), full(1, D),
                pl.BlockSpec((BR, D), lambda i: (i, 0))],
      out_specs=pl.BlockSpec((BR, D), lambda i: (i, 0)),
      out_shape=jax.ShapeDtypeStruct((N_PAD, D), jnp.float32),
  )(acc, acc, den, xl, xr, att, b, x)


def kernel(input, edge_index, Wl1, Wr1, att1, b1, Wl2, Wr2, att2, b2):
  src = edge_index[0].astype(jnp.int32)
  dst = edge_index[1].astype(jnp.int32)
  x = jnp.pad(input, ((0, N_PAD - N), (0, 0)))

  xl1, xr1 = _proj(x, Wl1, Wr1)
  acc_l1, den_l1 = _sc_edge(xl1, xr1, src, dst, att1)
  xl2, xr2 = _mid(acc_l1, den_l1, xl1, xr1,
                  att1.reshape(D, 1), b1.reshape(1, D), Wl2, Wr2)
  acc_l2, den_l2 = _sc_edge(xl2, xr2, src, dst, att2)
  out = _fin(acc_l2, den_l2, xl2, xr2,
             att2.reshape(D, 1), b2.reshape(1, D), x)
  return out[:N]


# speedup vs baseline: 1.1652x; 1.0210x over previous
import jax
import jax.numpy as jnp
from jax import lax
from jax.experimental import pallas as pl
from jax.experimental.pallas import tpu as pltpu
from jax.experimental.pallas import tpu_sc as plsc

N = 10000
N_PAD = 10240
D = 128
E = 320000
NEG = 0.2

NC, NS = 2, 16
NW = NC * NS
EPW = E // NW
C = 40
NCH = EPW // C
RPT = N_PAD // NS
LJ = D // 16
LANES = 16
GROUPS = ((0, 0), (16, 0), (24, 8))


def _lane_gather(x, idx):
  return lax.gather(
      x, idx[:, None],
      lax.GatherDimensionNumbers(offset_dims=(), collapsed_slice_dims=(0,),
                                 start_index_map=(0,)),
      slice_sizes=(1,),
      mode=lax.GatherScatterMode.PROMISE_IN_BOUNDS)


def _sc_edge_body(xl, xr, src, dst, att,
                  acc_out, den_out,
                  att_v,
                  si0, si1, di0, di1,
                  ra0, ra1, rb0, rb1, den_t,
                  smi0, smi1, smg0, smg1, smsc, acc_sh):
  sis, dis = [si0, si1], [di0, di1]
  ras, rbs = [ra0, ra1], [rb0, rb1]
  smis, smgs = [smi0, smi1], [smg0, smg1]
  cid = lax.axis_index("c")
  sid = lax.axis_index("s")
  wid = cid * NS + sid
  zero16 = jnp.zeros((LANES,), jnp.float32)
  zero16i = jnp.zeros((LANES,), jnp.int32)
  iota16 = jnp.arange(LANES, dtype=jnp.int32)

  def _zrow(i, carry):
    for j in range(LJ):
      ra0[i, pl.ds(16 * j, 16)] = zero16
    return carry
  lax.fori_loop(0, C, _zrow, 0)
  row0 = sid * RPT
  for k in range(RPT // C):
    pltpu.async_copy(ra0, acc_sh.at[pl.ds(row0 + k * C, C)], smsc)

  def _zden(i, carry):
    den_t[0, pl.ds(16 * i, 16)] = zero16
    return carry
  lax.fori_loop(0, N_PAD // 16, _zden, 0)

  pltpu.sync_copy(att, att_v)
  attv = [att_v[pl.ds(16 * j, 16)] for j in range(LJ)]
  for k in range(RPT // C):
    pltpu.make_async_copy(ra0, acc_sh.at[pl.ds(row0 + k * C, C)], smsc).wait()
  plsc.subcore_barrier()

  ebase = wid * EPW

  def fire_idx(k, p):
    base = ebase + k * C
    pltpu.async_copy(src.at[pl.ds(base, C)], sis[p], smis[p])
    pltpu.async_copy(dst.at[pl.ds(base, C)], dis[p], smis[p])

  def wait_idx(p):
    pltpu.make_async_copy(src.at[pl.ds(0, C)], sis[p], smis[p]).wait()
    pltpu.make_async_copy(dst.at[pl.ds(0, C)], dis[p], smis[p]).wait()

  def fire_gather(pi, pr):
    pltpu.async_copy(xl.at[sis[pi]], ras[pr], smgs[pr])
    pltpu.async_copy(xr.at[dis[pi]], rbs[pr], smgs[pr])

  def wait_gather(pi, pr):
    pltpu.make_async_copy(xl.at[sis[pi]], ras[pr], smgs[pr]).wait()
    pltpu.make_async_copy(xr.at[dis[pi]], rbs[pr], smgs[pr]).wait()

  def compute_chunk(pi, pr):
    ra, rb, dv = ras[pr], rbs[pr], dis[pi]

    @plsc.parallel_loop(0, C, step=1, unroll=4)
    def _edge(e):
      a = [ra[e, pl.ds(16 * j, 16)] for j in range(LJ)]
      acc = zero16
      for j in range(LJ):
        t = a[j] + rb[e, pl.ds(16 * j, 16)]
        acc = acc + attv[j] * jnp.maximum(t, NEG * t)
      sv = jnp.exp(jnp.broadcast_to(jnp.sum(acc), (LANES,)))
      for j in range(LJ):
        ra[e, pl.ds(16 * j, 16)] = a[j] * sv
      rb[e, pl.ds(0, 16)] = sv

    sc = pltpu.async_copy(ra, acc_sh.at[dv], smsc, add=True)

    for start, vfrom in GROUPS:
      did = dv[pl.ds(start, LANES)]
      svals = plsc.load_gather(rb, [iota16 + start, zero16i])
      if vfrom:
        svals = jnp.where(iota16 >= vfrom, svals, 0.0)
      ks, vs = plsc.sort_key_val(did, svals)
      for d in (1, 2, 4, 8):
        pidx = jnp.maximum(iota16 - d, 0)
        pk = _lane_gather(ks, pidx)
        pv = _lane_gather(vs, pidx)
        take = jnp.logical_and(iota16 >= d, pk == ks)
        vs = vs + jnp.where(take, pv, 0.0)
      nk = _lane_gather(ks, jnp.minimum(iota16 + 1, LANES - 1))
      is_last = jnp.logical_or(iota16 == LANES - 1, nk != ks)
      plsc.addupdate_scatter(den_t, [zero16i, ks], vs, mask=is_last)

    sc.wait()

  fire_idx(0, 0)
  fire_idx(1, 1)
  wait_idx(0)
  fire_gather(0, 0)
  wait_idx(1)
  fire_gather(1, 1)
  wait_gather(0, 0)
  compute_chunk(0, 0)
  fire_idx(2, 0)

  def _body(j, carry):
    for p in range(2):
      k = 1 + 2 * j + p
      cur, nxt = (1 + p) % 2, p
      wait_idx(nxt)
      fire_gather(nxt, nxt)
      wait_gather(cur, cur)
      compute_chunk(cur, cur)

      @pl.when(k + 2 < NCH)
      def _():
        fire_idx(k + 2, cur)
    return carry
  lax.fori_loop(0, (NCH - 2) // 2, _body, 0)

  wait_gather((NCH - 1) % 2, (NCH - 1) % 2)
  compute_chunk((NCH - 1) % 2, (NCH - 1) % 2)

  pltpu.async_copy(den_t, den_out.at[wid], smsc)
  plsc.subcore_barrier()

  def w_in(k, pr):
    pltpu.async_copy(acc_sh.at[pl.ds(row0 + k * C, C)], ras[pr], smis[pr])

  def w_in_wait(k, pr):
    pltpu.make_async_copy(acc_sh.at[pl.ds(row0 + k * C, C)], ras[pr],
                          smis[pr]).wait()

  def w_out(k, pr):
    pltpu.async_copy(ras[pr], acc_out.at[cid, pl.ds(row0 + k * C, C)],
                     smgs[pr])

  def w_out_wait(k, pr):
    pltpu.make_async_copy(ras[pr], acc_out.at[cid, pl.ds(row0 + k * C, C)],
                          smgs[pr]).wait()

  NWR = RPT // C
  w_in(0, 0)
  for k in range(NWR):
    pr = k % 2
    w_in_wait(k, pr)
    w_out(k, pr)
    if k + 1 < NWR:
      if k >= 1:
        w_out_wait(k - 1, (k - 1) % 2)
      w_in(k + 1, (k + 1) % 2)
  w_out_wait(NWR - 2, (NWR - 2) % 2)
  w_out_wait(NWR - 1, (NWR - 1) % 2)
  pltpu.make_async_copy(den_t, den_out.at[wid], smsc).wait()


_sc_edge = pl.kernel(
    _sc_edge_body,
    compiler_params=pltpu.CompilerParams(needs_layout_passes=False),
    out_type=(jax.ShapeDtypeStruct((NC, N_PAD, D), jnp.float32),
              jax.ShapeDtypeStruct((NW, 1, N_PAD), jnp.float32)),
    mesh=plsc.VectorSubcoreMesh(core_axis_name="c", subcore_axis_name="s"),
    scratch_types=(
        [pltpu.VMEM((D,), jnp.float32)]
        + [pltpu.VMEM((C,), jnp.int32)] * 4
        + [pltpu.VMEM((C, D), jnp.float32)] * 4
        + [pltpu.VMEM((1, N_PAD), jnp.float32)]
        + [pltpu.SemaphoreType.DMA] * 5
        + [pltpu.VMEM_SHARED((N_PAD, D), jnp.float32)]
    ),
)


BR = 1024


def _proj_body(x_ref, wl_ref, wr_ref, xl_ref, xr_ref):
  x = x_ref[...]
  xl_ref[...] = jnp.dot(x, wl_ref[...], preferred_element_type=jnp.float32)
  xr_ref[...] = jnp.dot(x, wr_ref[...], preferred_element_type=jnp.float32)


def _proj(x, wl, wr):
  return pl.pallas_call(
      _proj_body,
      grid=(N_PAD // BR,),
      in_specs=[pl.BlockSpec((BR, D), lambda i: (i, 0)),
                pl.BlockSpec((D, D), lambda i: (0, 0)),
                pl.BlockSpec((D, D), lambda i: (0, 0))],
      out_specs=[pl.BlockSpec((BR, D), lambda i: (i, 0))] * 2,
      out_shape=(jax.ShapeDtypeStruct((N_PAD, D), jnp.float32),) * 2,
  )(x, wl, wr)


def _x1_of(acc0, acc1, den32, xl, xr, att, b):
  t = xl + xr
  lr = jnp.maximum(t, NEG * t)
  s_self = jnp.exp(jnp.dot(lr, att, preferred_element_type=jnp.float32))
  den_n = lax.dot_general(den32, jnp.ones((NW, 1), jnp.float32),
                          (((0,), (0,)), ((), ())),
                          preferred_element_type=jnp.float32)
  dtot = den_n + s_self + 1e-16
  num = acc0 + acc1 + s_self * xl
  return num / dtot + b


def _acc_specs():
  return [pl.BlockSpec((1, BR, D), lambda i: (0, i, 0)),
          pl.BlockSpec((1, BR, D), lambda i: (1, i, 0)),
          pl.BlockSpec((NW, 1, BR), lambda i: (0, 0, i))]


def _mid_body(acc0_ref, acc1_ref, den_ref, xl_ref, xr_ref,
              att_ref, b_ref, wl2_ref, wr2_ref, xl2_ref, xr2_ref):
  x1 = _x1_of(acc0_ref[0], acc1_ref[0], den_ref[:, 0, :],
              xl_ref[...], xr_ref[...], att_ref[...], b_ref[...])
  xl2_ref[...] = jnp.dot(x1, wl2_ref[...], preferred_element_type=jnp.float32)
  xr2_ref[...] = jnp.dot(x1, wr2_ref[...], preferred_element_type=jnp.float32)


def _mid(acc, den, xl, xr, att, b, wl2, wr2):
  full = lambda r, c: pl.BlockSpec((r, c), lambda i: (0, 0))
  return pl.pallas_call(
      _mid_body,
      grid=(N_PAD // BR,),
      in_specs=_acc_specs() + [
                pl.BlockSpec((BR, D), lambda i: (i, 0)),
                pl.BlockSpec((BR, D), lambda i: (i, 0)),
                full(D, 1), full(1, D), full(D, D), full(D, D)],
      out_specs=[pl.BlockSpec((BR, D), lambda i: (i, 0))] * 2,
      out_shape=(jax.ShapeDtypeStruct((N_PAD, D), jnp.float32),) * 2,
  )(acc, acc, den, xl, xr, att, b, wl2, wr2)


def _fin_body(acc0_ref, acc1_ref, den_ref, xl_ref, xr_ref,
              att_ref, b_ref, x_ref, out_ref):
  x2 = _x1_of(acc0_ref[0], acc1_ref[0], den_ref[:, 0, :],
              xl_ref[...], xr_ref[...], att_ref[...], b_ref[...])
  out_ref[...] = (x2 + x_ref[...]) * 0.5


def _fin(acc, den, xl, xr, att, b, x):
  full = lambda r, c: pl.BlockSpec((r, c), lambda i: (0, 0))
  return pl.pallas_call(
      _fin_body,
      grid=(N_PAD // BR,),
      in_specs=_acc_specs() + [
                pl.BlockSpec((BR, D), lambda i: (i, 0)),
                pl.BlockSpec((BR, D), lambda i: (i, 0)),
                full(D, 1), full(1, D),
                pl.BlockSpec((BR, D), lambda i: (i, 0))],
      out_specs=pl.BlockSpec((BR, D), lambda i: (i, 0)),
      out_shape=jax.ShapeDtypeStruct((N_PAD, D), jnp.float32),
  )(acc, acc, den, xl, xr, att, b, x)


def kernel(input, edge_index, Wl1, Wr1, att1, b1, Wl2, Wr2, att2, b2):
  src = edge_index[0].astype(jnp.int32)
  dst = edge_index[1].astype(jnp.int32)
  x = jnp.pad(input, ((0, N_PAD - N), (0, 0)))

  xl1, xr1 = _proj(x, Wl1, Wr1)
  acc_l1, den_l1 = _sc_edge(xl1, xr1, src, dst, att1)
  xl2, xr2 = _mid(acc_l1, den_l1, xl1, xr1,
                  att1.reshape(D, 1), b1.reshape(1, D), Wl2, Wr2)
  acc_l2, den_l2 = _sc_edge(xl2, xr2, src, dst, att2)
  out = _fin(acc_l2, den_l2, xl2, xr2,
             att2.reshape(D, 1), b2.reshape(1, D), x)
  return out[:N]
